# baseline (speedup 1.0000x reference)
import jax
import jax.numpy as jnp
from jax.experimental import pallas as pl

_SELU_SCALE = 1.0507009873554805
_SELU_ALPHA = 1.6732632423543772

_BLOCK_B = 128


def _fused_kernel(emb_ref, cat_ref, prem_ref, sent_ref, temp_ref, week_ref,
                  hour_ref, w1_ref, b1_ref, w2_ref, b2_ref,
                  ptab_ref, stab_ref, ttab_ref, wtab_ref, htab_ref, out_ref):
    RB, L, ART = emb_ref.shape
    lookups = ((prem_ref, ptab_ref), (sent_ref, stab_ref),
               (temp_ref, ttab_ref), (week_ref, wtab_ref),
               (hour_ref, htab_ref))
    w1a = w1_ref[:ART, :]
    w1b = w1_ref[ART:, :]
    for l in range(L):
        h = jnp.dot(emb_ref[:, l, :], w1a, preferred_element_type=jnp.float32)
        h += jnp.dot(cat_ref[:, l, :], w1b, preferred_element_type=jnp.float32)
        h += b1_ref[...]
        h = _SELU_SCALE * jnp.where(h > 0, h, _SELU_ALPHA * (jnp.exp(h) - 1.0))
        x = jnp.dot(h.astype(jnp.bfloat16), w2_ref[...],
                    preferred_element_type=jnp.float32)
        x += b2_ref[...]
        for idx_ref, tab_ref in lookups:
            k = tab_ref.shape[0]
            iota = jax.lax.broadcasted_iota(jnp.int32, (1, k), 1)
            oh = (idx_ref[:, l:l + 1] == iota).astype(jnp.float32)
            x += jnp.dot(oh, tab_ref[...], preferred_element_type=jnp.float32)
        out_ref[:, l, :] = x


def kernel(embs, cat_embs, premium, sentiment, mask, temporal, weekdays, hours,
           W1, b1, W2, b2, premium_tab, sentiment_tab, temporal_tab,
           weekday_tab, hour_tab):
    B, L, ART = embs.shape
    CAT = cat_embs.shape[2]
    DIMS = W2.shape[1]
    RB = _BLOCK_B
    grid = B // RB

    idx_spec = pl.BlockSpec((RB, L), lambda i: (i, 0))

    def tab_spec(t):
        return pl.BlockSpec(t.shape, lambda i: (0, 0))

    out = pl.pallas_call(
        _fused_kernel,
        grid=(grid,),
        in_specs=[
            pl.BlockSpec((RB, L, ART), lambda i: (i, 0, 0)),
            pl.BlockSpec((RB, L, CAT), lambda i: (i, 0, 0)),
            idx_spec, idx_spec, idx_spec, idx_spec, idx_spec,
            pl.BlockSpec((ART + CAT, DIMS), lambda i: (0, 0)),
            pl.BlockSpec((1, DIMS), lambda i: (0, 0)),
            pl.BlockSpec((DIMS, DIMS), lambda i: (0, 0)),
            pl.BlockSpec((1, DIMS), lambda i: (0, 0)),
            tab_spec(premium_tab), tab_spec(sentiment_tab),
            tab_spec(temporal_tab), tab_spec(weekday_tab), tab_spec(hour_tab),
        ],
        out_specs=pl.BlockSpec((RB, L, DIMS), lambda i: (i, 0, 0)),
        out_shape=jax.ShapeDtypeStruct((B, L, DIMS), jnp.float32),
    )(embs.astype(jnp.bfloat16), cat_embs.astype(jnp.bfloat16),
      premium, sentiment, temporal, weekdays, hours,
      W1.astype(jnp.bfloat16), b1.reshape(1, DIMS),
      W2.astype(jnp.bfloat16), b2.reshape(1, DIMS),
      premium_tab, sentiment_tab, temporal_tab, weekday_tab, hour_tab)

    return (out, mask)


# speedup vs baseline: 5.3646x; 5.3646x over previous
import jax
import jax.numpy as jnp
from jax.experimental import pallas as pl

_SELU_SCALE = 1.0507009873554805
_SELU_ALPHA = 1.6732632423543772

_OFFSETS = (0, 2, 5, 37, 44)
_TAB_ROWS = 68
_TAB_PAD = 128

_BLOCK_B = 1024


def _fused_kernel(emb_ref, cat_ref, prem_ref, sent_ref, temp_ref, week_ref,
                  hour_ref, w1t_ref, b1_ref, w2_ref, b2_ref, tab_ref, out_ref):
    ART = emb_ref.shape[2]
    emb = emb_ref[0]
    cat = cat_ref[0]
    w1at = w1t_ref[:, :ART]
    w1bt = w1t_ref[:, ART:]
    ht = jax.lax.dot_general(w1at, emb, (((1,), (1,)), ((), ())),
                             preferred_element_type=jnp.float32)
    ht += jax.lax.dot_general(w1bt, cat, (((1,), (1,)), ((), ())),
                              preferred_element_type=jnp.float32)
    ht += b1_ref[...]
    ht = _SELU_SCALE * jnp.where(ht > 0, ht, _SELU_ALPHA * (jnp.exp(ht) - 1.0))
    xt = jax.lax.dot_general(w2_ref[...], ht, (((0,), (0,)), ((), ())),
                             preferred_element_type=jnp.float32)
    xt += b2_ref[...]
    iota = jax.lax.broadcasted_iota(jnp.int32, (_TAB_PAD, 1), 0)
    oht = jnp.zeros((_TAB_PAD, emb.shape[0]), dtype=jnp.float32)
    idx_refs = (prem_ref, sent_ref, temp_ref, week_ref, hour_ref)
    for t in range(5):
        idx = idx_refs[t][0] + _OFFSETS[t]
        oht += (idx == iota).astype(jnp.float32)
    xt += jax.lax.dot_general(tab_ref[...], oht, (((0,), (0,)), ((), ())),
                              preferred_element_type=jnp.float32)
    out_ref[0] = xt


def kernel(embs, cat_embs, premium, sentiment, mask, temporal, weekdays, hours,
           W1, b1, W2, b2, premium_tab, sentiment_tab, temporal_tab,
           weekday_tab, hour_tab):
    B, L, ART = embs.shape
    CAT = cat_embs.shape[2]
    DIMS = W2.shape[1]
    RB = _BLOCK_B
    gb = B // RB

    emb_t = jnp.transpose(embs, (1, 0, 2))
    cat_t = jnp.transpose(cat_embs, (1, 0, 2))

    def idx3(a):
        return jnp.transpose(a, (1, 0)).reshape(L, 1, B)

    tab = jnp.concatenate([premium_tab, sentiment_tab, temporal_tab,
                           weekday_tab, hour_tab,
                           jnp.zeros((_TAB_PAD - _TAB_ROWS, DIMS), jnp.float32)])

    fspec = lambda F: pl.BlockSpec((1, RB, F), lambda l, j: (l, j, 0))
    ispec = pl.BlockSpec((1, 1, RB), lambda l, j: (l, 0, j))
    wspec = lambda a, b: pl.BlockSpec((a, b), lambda l, j: (0, 0))

    out = pl.pallas_call(
        _fused_kernel,
        grid=(L, gb),
        in_specs=[
            fspec(ART), fspec(CAT),
            ispec, ispec, ispec, ispec, ispec,
            wspec(DIMS, ART + CAT), wspec(DIMS, 1),
            wspec(DIMS, DIMS), wspec(DIMS, 1), wspec(_TAB_PAD, DIMS),
        ],
        out_specs=pl.BlockSpec((1, DIMS, RB), lambda l, j: (l, 0, j)),
        out_shape=jax.ShapeDtypeStruct((L, DIMS, B), jnp.float32),
    )(emb_t, cat_t, idx3(premium), idx3(sentiment), idx3(temporal),
      idx3(weekdays), idx3(hours),
      W1.T, b1.reshape(DIMS, 1), W2, b2.reshape(DIMS, 1), tab)

    return (jnp.transpose(out, (2, 0, 1)), mask)


# speedup vs baseline: 5.6357x; 1.0505x over previous
import jax
import jax.numpy as jnp
from jax.experimental import pallas as pl

_SELU_SCALE = 1.0507009873554805
_SELU_ALPHA = 1.6732632423543772

_BLOCK_B = 1024


def _fused_kernel(emb_ref, cat_ref, idx_ref, w1t_ref, b1_ref, w2_ref, b2_ref,
                  ptab_ref, stab_ref, ttab_ref, wtab_ref, htab_ref, out_ref):
    ART = emb_ref.shape[2]
    emb = emb_ref[0]
    cat = cat_ref[0]
    w1at = w1t_ref[:, :ART]
    w1bt = w1t_ref[:, ART:]
    ht = jax.lax.dot_general(w1at, emb, (((1,), (1,)), ((), ())),
                             preferred_element_type=jnp.float32)
    ht += jax.lax.dot_general(w1bt, cat, (((1,), (1,)), ((), ())),
                              preferred_element_type=jnp.float32)
    ht += b1_ref[...]
    ht = _SELU_SCALE * jnp.where(ht > 0, ht, _SELU_ALPHA * (jnp.exp(ht) - 1.0))
    xt = jax.lax.dot_general(w2_ref[...], ht, (((0,), (0,)), ((), ())),
                             preferred_element_type=jnp.float32)
    xt += b2_ref[...]
    for t, tab_ref in enumerate((ptab_ref, stab_ref, ttab_ref, wtab_ref,
                                 htab_ref)):
        k = tab_ref.shape[0]
        iota = jax.lax.broadcasted_iota(jnp.int32, (k, 1), 0)
        oht = (idx_ref[0, t:t + 1, :] == iota).astype(jnp.float32)
        xt += jax.lax.dot_general(tab_ref[...], oht, (((0,), (0,)), ((), ())),
                                  preferred_element_type=jnp.float32)
    out_ref[0] = xt


def kernel(embs, cat_embs, premium, sentiment, mask, temporal, weekdays, hours,
           W1, b1, W2, b2, premium_tab, sentiment_tab, temporal_tab,
           weekday_tab, hour_tab):
    B, L, ART = embs.shape
    CAT = cat_embs.shape[2]
    DIMS = W2.shape[1]
    RB = _BLOCK_B
    gb = B // RB

    emb_t = jnp.transpose(embs, (1, 0, 2))
    cat_t = jnp.transpose(cat_embs, (1, 0, 2))
    idx_all = jnp.stack([premium.T, sentiment.T, temporal.T, weekdays.T,
                         hours.T], axis=1)

    fspec = lambda F: pl.BlockSpec((1, RB, F), lambda l, j: (l, j, 0))
    wspec = lambda a, b: pl.BlockSpec((a, b), lambda l, j: (0, 0))

    out = pl.pallas_call(
        _fused_kernel,
        grid=(L, gb),
        in_specs=[
            fspec(ART), fspec(CAT),
            pl.BlockSpec((1, 5, RB), lambda l, j: (l, 0, j)),
            wspec(DIMS, ART + CAT), wspec(DIMS, 1),
            wspec(DIMS, DIMS), wspec(DIMS, 1),
            wspec(2, DIMS), wspec(3, DIMS), wspec(32, DIMS),
            wspec(7, DIMS), wspec(24, DIMS),
        ],
        out_specs=pl.BlockSpec((1, DIMS, RB), lambda l, j: (l, 0, j)),
        out_shape=jax.ShapeDtypeStruct((L, DIMS, B), jnp.float32),
    )(emb_t, cat_t, idx_all,
      W1.T, b1.reshape(DIMS, 1), W2, b2.reshape(DIMS, 1),
      premium_tab, sentiment_tab, temporal_tab, weekday_tab, hour_tab)

    return (jnp.transpose(out, (2, 0, 1)), mask)


# speedup vs baseline: 6.8779x; 1.2204x over previous
import jax
import jax.numpy as jnp
from jax.experimental import pallas as pl

_SELU_SCALE = 1.0507009873554805
_SELU_ALPHA = 1.6732632423543772

_BLOCK_B = 2048


def _fused_kernel(emb_ref, cat_ref, idx_ref, w1t_ref, b1_ref, w2_ref, b2_ref,
                  ptab_ref, stab_ref, ttab_ref, wtab_ref, htab_ref, out_ref):
    ART = emb_ref.shape[2]
    emb = emb_ref[0]
    cat = cat_ref[0]
    w1at = w1t_ref[:, :ART]
    w1bt = w1t_ref[:, ART:]
    ht = jax.lax.dot_general(w1at, emb, (((1,), (1,)), ((), ())),
                             preferred_element_type=jnp.float32)
    ht += jax.lax.dot_general(w1bt, cat, (((1,), (1,)), ((), ())),
                              preferred_element_type=jnp.float32)
    ht += b1_ref[...]
    ht = _SELU_SCALE * jnp.where(ht > 0, ht, _SELU_ALPHA * (jnp.exp(ht) - 1.0))
    xt = jax.lax.dot_general(w2_ref[...], ht, (((0,), (0,)), ((), ())),
                             preferred_element_type=jnp.float32)
    xt += b2_ref[...]
    for t, tab_ref in enumerate((ptab_ref, stab_ref, ttab_ref, wtab_ref,
                                 htab_ref)):
        k = tab_ref.shape[0]
        iota = jax.lax.broadcasted_iota(jnp.int32, (k, 1), 0)
        oht = (idx_ref[0, t:t + 1, :] == iota).astype(jnp.float32)
        xt += jax.lax.dot_general(tab_ref[...], oht, (((0,), (0,)), ((), ())),
                                  preferred_element_type=jnp.float32)
    out_ref[0] = xt


def kernel(embs, cat_embs, premium, sentiment, mask, temporal, weekdays, hours,
           W1, b1, W2, b2, premium_tab, sentiment_tab, temporal_tab,
           weekday_tab, hour_tab):
    B, L, ART = embs.shape
    CAT = cat_embs.shape[2]
    DIMS = W2.shape[1]
    RB = _BLOCK_B
    gb = B // RB

    emb_t = jnp.transpose(embs, (1, 0, 2))
    cat_t = jnp.transpose(cat_embs, (1, 0, 2))
    idx_all = jnp.stack([premium.T, sentiment.T, temporal.T, weekdays.T,
                         hours.T], axis=1)

    fspec = lambda F: pl.BlockSpec((1, RB, F), lambda l, j: (l, j, 0))
    wspec = lambda a, b: pl.BlockSpec((a, b), lambda l, j: (0, 0))

    out = pl.pallas_call(
        _fused_kernel,
        grid=(L, gb),
        in_specs=[
            fspec(ART), fspec(CAT),
            pl.BlockSpec((1, 5, RB), lambda l, j: (l, 0, j)),
            wspec(DIMS, ART + CAT), wspec(DIMS, 1),
            wspec(DIMS, DIMS), wspec(DIMS, 1),
            wspec(2, DIMS), wspec(3, DIMS), wspec(32, DIMS),
            wspec(7, DIMS), wspec(24, DIMS),
        ],
        out_specs=pl.BlockSpec((1, DIMS, RB), lambda l, j: (l, 0, j)),
        out_shape=jax.ShapeDtypeStruct((L, DIMS, B), jnp.float32),
    )(emb_t, cat_t, idx_all,
      W1.T, b1.reshape(DIMS, 1), W2, b2.reshape(DIMS, 1),
      premium_tab, sentiment_tab, temporal_tab, weekday_tab, hour_tab)

    return (jnp.transpose(out, (2, 0, 1)), mask)


# speedup vs baseline: 7.5556x; 1.0985x over previous
import jax
import jax.numpy as jnp
from jax.experimental import pallas as pl

_SELU_SCALE = 1.0507009873554805
_SELU_ALPHA = 1.6732632423543772

_BLOCK_B = 4096


def _fused_kernel(emb_ref, cat_ref, idx_ref, w1t_ref, b1_ref, w2_ref, b2_ref,
                  ptab_ref, stab_ref, ttab_ref, wtab_ref, htab_ref, out_ref):
    ART = emb_ref.shape[2]
    emb = emb_ref[0]
    cat = cat_ref[0]
    w1at = w1t_ref[:, :ART]
    w1bt = w1t_ref[:, ART:]
    ht = jax.lax.dot_general(w1at, emb, (((1,), (1,)), ((), ())),
                             preferred_element_type=jnp.float32)
    ht += jax.lax.dot_general(w1bt, cat, (((1,), (1,)), ((), ())),
                              preferred_element_type=jnp.float32)
    ht += b1_ref[...]
    ht = _SELU_SCALE * jnp.where(ht > 0, ht, _SELU_ALPHA * (jnp.exp(ht) - 1.0))
    xt = jax.lax.dot_general(w2_ref[...], ht, (((0,), (0,)), ((), ())),
                             preferred_element_type=jnp.float32)
    xt += b2_ref[...]
    for t, tab_ref in enumerate((ptab_ref, stab_ref, ttab_ref, wtab_ref,
                                 htab_ref)):
        k = tab_ref.shape[0]
        iota = jax.lax.broadcasted_iota(jnp.int32, (k, 1), 0)
        oht = (idx_ref[0, t:t + 1, :] == iota).astype(jnp.float32)
        xt += jax.lax.dot_general(tab_ref[...], oht, (((0,), (0,)), ((), ())),
                                  preferred_element_type=jnp.float32)
    out_ref[0] = xt


def kernel(embs, cat_embs, premium, sentiment, mask, temporal, weekdays, hours,
           W1, b1, W2, b2, premium_tab, sentiment_tab, temporal_tab,
           weekday_tab, hour_tab):
    B, L, ART = embs.shape
    CAT = cat_embs.shape[2]
    DIMS = W2.shape[1]
    RB = _BLOCK_B
    gb = B // RB

    emb_t = jnp.transpose(embs, (1, 0, 2))
    cat_t = jnp.transpose(cat_embs, (1, 0, 2))
    idx_all = jnp.stack([premium.T, sentiment.T, temporal.T, weekdays.T,
                         hours.T], axis=1)

    fspec = lambda F: pl.BlockSpec((1, RB, F), lambda l, j: (l, j, 0))
    wspec = lambda a, b: pl.BlockSpec((a, b), lambda l, j: (0, 0))

    out = pl.pallas_call(
        _fused_kernel,
        grid=(L, gb),
        in_specs=[
            fspec(ART), fspec(CAT),
            pl.BlockSpec((1, 5, RB), lambda l, j: (l, 0, j)),
            wspec(DIMS, ART + CAT), wspec(DIMS, 1),
            wspec(DIMS, DIMS), wspec(DIMS, 1),
            wspec(2, DIMS), wspec(3, DIMS), wspec(32, DIMS),
            wspec(7, DIMS), wspec(24, DIMS),
        ],
        out_specs=pl.BlockSpec((1, DIMS, RB), lambda l, j: (l, 0, j)),
        out_shape=jax.ShapeDtypeStruct((L, DIMS, B), jnp.float32),
    )(emb_t, cat_t, idx_all,
      W1.T, b1.reshape(DIMS, 1), W2, b2.reshape(DIMS, 1),
      premium_tab, sentiment_tab, temporal_tab, weekday_tab, hour_tab)

    return (jnp.transpose(out, (2, 0, 1)), mask)


# speedup vs baseline: 8.2353x; 1.0900x over previous
import jax
import jax.numpy as jnp
from jax.experimental import pallas as pl

_SELU_SCALE = 1.0507009873554805
_SELU_ALPHA = 1.6732632423543772

_BLOCK_B = 4096

_SHIFTS = (0, 1, 3, 8, 11)
_MASKS = (1, 3, 31, 7, 31)


def _fused_kernel(emb_ref, cat_ref, idx_ref, w1t_ref, b1_ref, w2_ref,
                  ptab_ref, stab_ref, ttab_ref, wtab_ref, htab_ref, out_ref):
    ART = emb_ref.shape[2]
    emb = emb_ref[0]
    cat = cat_ref[0]
    w1at = w1t_ref[:, :ART]
    w1bt = w1t_ref[:, ART:]
    ht = jax.lax.dot_general(w1at, emb, (((1,), (1,)), ((), ())),
                             preferred_element_type=jnp.float32)
    ht += jax.lax.dot_general(w1bt, cat, (((1,), (1,)), ((), ())),
                              preferred_element_type=jnp.float32)
    ht += b1_ref[...]
    ht = _SELU_SCALE * jnp.where(ht > 0, ht, _SELU_ALPHA * (jnp.exp(ht) - 1.0))
    xt = jax.lax.dot_general(w2_ref[...], ht, (((0,), (0,)), ((), ())),
                             preferred_element_type=jnp.float32)
    packed = idx_ref[0]
    for t, tab_ref in enumerate((ptab_ref, stab_ref, ttab_ref, wtab_ref,
                                 htab_ref)):
        k = tab_ref.shape[0]
        idx = (packed >> _SHIFTS[t]) & _MASKS[t]
        iota = jax.lax.broadcasted_iota(jnp.int32, (k, 1), 0)
        oht = (idx == iota).astype(jnp.float32)
        xt += jax.lax.dot_general(tab_ref[...], oht, (((0,), (0,)), ((), ())),
                                  preferred_element_type=jnp.float32)
    out_ref[0] = xt


def kernel(embs, cat_embs, premium, sentiment, mask, temporal, weekdays, hours,
           W1, b1, W2, b2, premium_tab, sentiment_tab, temporal_tab,
           weekday_tab, hour_tab):
    B, L, ART = embs.shape
    CAT = cat_embs.shape[2]
    DIMS = W2.shape[1]
    RB = _BLOCK_B
    gb = B // RB

    emb_t = jnp.transpose(embs, (1, 0, 2))
    cat_t = jnp.transpose(cat_embs, (1, 0, 2))

    packed = (premium.astype(jnp.int32)
              | (sentiment.astype(jnp.int32) << _SHIFTS[1])
              | (temporal.astype(jnp.int32) << _SHIFTS[2])
              | (weekdays.astype(jnp.int32) << _SHIFTS[3])
              | (hours.astype(jnp.int32) << _SHIFTS[4]))
    packed = packed.T.reshape(L, 1, B)

    ptab_eff = premium_tab + b2[None, :]

    fspec = lambda F: pl.BlockSpec((1, RB, F), lambda l, j: (l, j, 0))
    wspec = lambda a, b: pl.BlockSpec((a, b), lambda l, j: (0, 0))

    out = pl.pallas_call(
        _fused_kernel,
        grid=(L, gb),
        in_specs=[
            fspec(ART), fspec(CAT),
            pl.BlockSpec((1, 1, RB), lambda l, j: (l, 0, j)),
            wspec(DIMS, ART + CAT), wspec(DIMS, 1), wspec(DIMS, DIMS),
            wspec(2, DIMS), wspec(3, DIMS), wspec(32, DIMS),
            wspec(7, DIMS), wspec(24, DIMS),
        ],
        out_specs=pl.BlockSpec((1, DIMS, RB), lambda l, j: (l, 0, j)),
        out_shape=jax.ShapeDtypeStruct((L, DIMS, B), jnp.float32),
    )(emb_t, cat_t, packed,
      W1.T, b1.reshape(DIMS, 1), W2,
      ptab_eff, sentiment_tab, temporal_tab, weekday_tab, hour_tab)

    return (jnp.transpose(out, (2, 0, 1)), mask)
